# Initial kernel scaffold; baseline (speedup 1.0000x reference)
import jax
import jax.numpy as jnp
from jax.experimental import pallas as pl


def kernel(x, edge_index, W1_l, b1_l, W1_r, W2_l, b2_l, W2_r):
    raise NotImplementedError("write your pallas kernel here")


# speedup vs baseline: 4.4909x; 4.4909x over previous
import jax
import jax.numpy as jnp
from jax import lax
from jax.experimental import pallas as pl
from jax.experimental.pallas import tpu as pltpu
from jax.experimental.pallas import tpu_sc as plsc

N = 10000
E = 320000
D_IN = 128
HIDDEN = 128
N_CLASSES = 64

NC = 2
NS = 16
NW = NC * NS
EPW = E // NW
C = 80
NCHUNK = EPW // C
RPT = 640

F32 = jnp.float32


def _tc_mm2(x, wl, wr, d_out):
    def body(x_ref, wl_ref, wr_ref, y_ref, r_ref):
        xb = x_ref[...]
        y_ref[...] = jnp.dot(xb, wl_ref[...], preferred_element_type=F32)
        r_ref[...] = jnp.dot(xb, wr_ref[...], preferred_element_type=F32)

    return pl.pallas_call(
        body,
        out_shape=[jax.ShapeDtypeStruct((N, d_out), F32)] * 2,
    )(x, wl, wr)


def _tc_layer1_combine(accp, cntp, r1, b1, w2l, w2r):
    def body(a_ref, c_ref, r_ref, b_ref, wl_ref, wr_ref, yr_ref):
        a = a_ref[...]
        cc = c_ref[...]
        inv = 1.0 / jnp.maximum(cc[0, :, 0:1] + cc[1, :, 0:1], 1.0)
        h = (a[0] + a[1]) * inv + b_ref[...][None, :] + r_ref[...]
        h = jnp.maximum(h, 0.0)
        yr_ref[:, 0:N_CLASSES] = jnp.dot(h, wl_ref[...],
                                         preferred_element_type=F32)
        yr_ref[:, N_CLASSES:2 * N_CLASSES] = jnp.dot(
            h, wr_ref[...], preferred_element_type=F32)

    return pl.pallas_call(
        body,
        out_shape=jax.ShapeDtypeStruct((N, 2 * N_CLASSES), F32),
    )(accp, cntp, r1, b1, w2l, w2r)


def _tc_final(acc2p, cntp, yr2, b2):
    def body(a_ref, c_ref, yr_ref, b_ref, o_ref):
        a = a_ref[...]
        cc = c_ref[...]
        inv = 1.0 / jnp.maximum(cc[0, :, 0:1] + cc[1, :, 0:1], 1.0)
        agg = a[0, :, 0:N_CLASSES] + a[1, :, 0:N_CLASSES]
        r2 = yr_ref[:, N_CLASSES:2 * N_CLASSES]
        o = agg * inv + b_ref[...][None, :] + r2
        m = jnp.max(o, axis=1, keepdims=True)
        s = jnp.sum(jnp.exp(o - m), axis=1, keepdims=True)
        o_ref[...] = o - m - jnp.log(s)

    return pl.pallas_call(
        body,
        out_shape=jax.ShapeDtypeStruct((N, N_CLASSES), F32),
    )(acc2p, cntp, yr2, b2)


_MESH = plsc.VectorSubcoreMesh(
    core_axis_name="c", subcore_axis_name="s",
    num_cores=NC, num_subcores=NS)


def _make_sc_segsum(d):
    def body(y_hbm, src_hbm, dst_hbm, zacc_hbm, acc_out,
             src_v, dst_v, rows_v, acc_sh, sem):
        c = lax.axis_index("c")
        s = lax.axis_index("s")
        wid = c * NS + s
        base = pl.multiple_of(jnp.minimum(s * 624, N - RPT), 8)

        pltpu.sync_copy(zacc_hbm.at[pl.ds(base, RPT)],
                        acc_sh.at[pl.ds(base, RPT)])
        plsc.subcore_barrier()

        def step(j, carry):
            pltpu.sync_copy(src_hbm.at[wid].at[j], src_v)
            pltpu.sync_copy(dst_hbm.at[wid].at[j], dst_v)
            pltpu.async_copy(y_hbm.at[src_v.at[0]], rows_v, sem).wait()
            pltpu.sync_copy(rows_v, acc_sh.at[dst_v.at[0]], add=True)
            return carry
        lax.fori_loop(0, NCHUNK, step, 0)

        plsc.subcore_barrier()
        pltpu.sync_copy(acc_sh.at[pl.ds(base, RPT)],
                        acc_out.at[c].at[pl.ds(base, RPT)])

    return pl.kernel(
        body,
        out_type=[jax.ShapeDtypeStruct((NC, N, d), F32)],
        mesh=_MESH,
        scratch_types=[
            pltpu.VMEM((1, C), jnp.int32),
            pltpu.VMEM((1, C), jnp.int32),
            pltpu.VMEM((C, d), F32),
            pltpu.VMEM_SHARED((N, d), F32),
            pltpu.SemaphoreType.DMA,
        ],
    )


def _make_sc_count():
    def body(dst_hbm, zacc_hbm, cnt_out, dst_v, ones_v, cnt_sh, sem):
        c = lax.axis_index("c")
        s = lax.axis_index("s")
        wid = c * NS + s
        base = pl.multiple_of(jnp.minimum(s * 624, N - RPT), 8)

        pltpu.sync_copy(zacc_hbm.at[pl.ds(base, RPT)],
                        cnt_sh.at[pl.ds(base, RPT)])
        one16 = jnp.ones((16,), F32)

        def orow(r, carry):
            for l in range(8):
                ones_v[r, pl.ds(l * 16, 16)] = one16
            return carry
        lax.fori_loop(0, C, orow, 0)
        plsc.subcore_barrier()

        def step(j, carry):
            pltpu.sync_copy(dst_hbm.at[wid].at[j], dst_v)
            pltpu.sync_copy(ones_v, cnt_sh.at[dst_v.at[0]], add=True)
            return carry
        lax.fori_loop(0, NCHUNK, step, 0)

        plsc.subcore_barrier()
        pltpu.sync_copy(cnt_sh.at[pl.ds(base, RPT)],
                        cnt_out.at[c].at[pl.ds(base, RPT)])

    return pl.kernel(
        body,
        out_type=[jax.ShapeDtypeStruct((NC, N, HIDDEN), F32)],
        mesh=_MESH,
        scratch_types=[
            pltpu.VMEM((1, C), jnp.int32),
            pltpu.VMEM((C, HIDDEN), F32),
            pltpu.VMEM_SHARED((N, HIDDEN), F32),
            pltpu.SemaphoreType.DMA,
        ],
    )


_sc_seg_h = _make_sc_segsum(HIDDEN)
_sc_seg_o = _make_sc_segsum(2 * N_CLASSES)
_sc_count = _make_sc_count()


@jax.jit
def kernel(x, edge_index, W1_l, b1_l, W1_r, W2_l, b2_l, W2_r):
    src3 = edge_index[0].reshape(NW, NCHUNK, 1, C)
    dst3 = edge_index[1].reshape(NW, NCHUNK, 1, C)

    y1, r1 = _tc_mm2(x, W1_l, W1_r, HIDDEN)
    zacc1 = jnp.zeros((N, HIDDEN), F32)
    cntp, = _sc_count(dst3, zacc1)
    acc1p, = _sc_seg_h(y1, src3, dst3, zacc1)

    yr2 = _tc_layer1_combine(acc1p, cntp, r1, b1_l, W2_l, W2_r)
    zacc2 = jnp.zeros((N, 2 * N_CLASSES), F32)
    acc2p, = _sc_seg_o(yr2, src3, dst3, zacc2)

    return _tc_final(acc2p, cntp, yr2, b2_l)


# speedup vs baseline: 8.1033x; 1.8044x over previous
import jax
import jax.numpy as jnp
from jax import lax
from jax.experimental import pallas as pl
from jax.experimental.pallas import tpu as pltpu
from jax.experimental.pallas import tpu_sc as plsc

N = 10000
E = 320000
D_IN = 128
HIDDEN = 128
N_CLASSES = 64

NC = 2
NS = 16
NW = NC * NS
EPW = E // NW
C = 80
NCHUNK = EPW // C
RPT = 640

F32 = jnp.float32


def _tc_mm2(x, wl, wr, d_out):
    def body(x_ref, wl_ref, wr_ref, y_ref, r_ref):
        xb = x_ref[...]
        y_ref[...] = jnp.dot(xb, wl_ref[...], preferred_element_type=F32)
        r_ref[...] = jnp.dot(xb, wr_ref[...], preferred_element_type=F32)

    return pl.pallas_call(
        body,
        out_shape=[jax.ShapeDtypeStruct((N, d_out), F32)] * 2,
    )(x, wl, wr)


def _tc_layer1_combine(accp, cntp, r1, b1, w2l, w2r):
    def body(a_ref, c_ref, r_ref, b_ref, wl_ref, wr_ref, yr_ref):
        a = a_ref[...]
        cc = c_ref[...]
        inv = 1.0 / jnp.maximum(cc[0, :, 0:1] + cc[1, :, 0:1], 1.0)
        h = (a[0] + a[1]) * inv + b_ref[...][None, :] + r_ref[...]
        h = jnp.maximum(h, 0.0)
        yr_ref[:, 0:N_CLASSES] = jnp.dot(h, wl_ref[...],
                                         preferred_element_type=F32)
        yr_ref[:, N_CLASSES:2 * N_CLASSES] = jnp.dot(
            h, wr_ref[...], preferred_element_type=F32)

    return pl.pallas_call(
        body,
        out_shape=jax.ShapeDtypeStruct((N, 2 * N_CLASSES), F32),
    )(accp, cntp, r1, b1, w2l, w2r)


def _tc_final(acc2p, cntp, yr2, b2):
    def body(a_ref, c_ref, yr_ref, b_ref, o_ref):
        a = a_ref[...]
        cc = c_ref[...]
        inv = 1.0 / jnp.maximum(cc[0, :, 0:1] + cc[1, :, 0:1], 1.0)
        agg = a[0, :, 0:N_CLASSES] + a[1, :, 0:N_CLASSES]
        r2 = yr_ref[:, N_CLASSES:2 * N_CLASSES]
        o = agg * inv + b_ref[...][None, :] + r2
        m = jnp.max(o, axis=1, keepdims=True)
        s = jnp.sum(jnp.exp(o - m), axis=1, keepdims=True)
        o_ref[...] = o - m - jnp.log(s)

    return pl.pallas_call(
        body,
        out_shape=jax.ShapeDtypeStruct((N, N_CLASSES), F32),
    )(acc2p, cntp, yr2, b2)


_MESH = plsc.VectorSubcoreMesh(
    core_axis_name="c", subcore_axis_name="s",
    num_cores=NC, num_subcores=NS)


def _make_sc_segsum(d):
    def body(y_hbm, src_hbm, dst_hbm, zacc_hbm, acc_out,
             src0, dst0, rows0, src1, dst1, rows1, acc_sh,
             si0, sg0, si1, sg1):
        c = lax.axis_index("c")
        s = lax.axis_index("s")
        wid = c * NS + s
        base = pl.multiple_of(jnp.minimum(s * 624, N - RPT), 8)

        pltpu.sync_copy(zacc_hbm.at[pl.ds(base, RPT)],
                        acc_sh.at[pl.ds(base, RPT)])
        plsc.subcore_barrier()

        A = (src0, dst0, rows0, si0, sg0)
        B = (src1, dst1, rows1, si1, sg1)

        def start_idx(j, buf):
            s_, d_, _, si, _ = buf
            pltpu.async_copy(src_hbm.at[wid].at[j], s_, si)
            pltpu.async_copy(dst_hbm.at[wid].at[j], d_, si)

        def wait_idx(buf):
            s_, d_, _, si, _ = buf
            pltpu.make_async_copy(src_hbm.at[wid].at[0], s_, si).wait()
            pltpu.make_async_copy(dst_hbm.at[wid].at[0], d_, si).wait()

        def start_gather(buf):
            s_, _, r_, _, sg = buf
            pltpu.async_copy(y_hbm.at[s_.at[0]], r_, sg)

        def wait_gather(buf):
            s_, _, r_, _, sg = buf
            pltpu.make_async_copy(y_hbm.at[s_.at[0]], r_, sg).wait()

        def scatter(buf):
            _, d_, r_, _, _ = buf
            pltpu.sync_copy(r_, acc_sh.at[d_.at[0]], add=True)

        start_idx(0, A)
        start_idx(1, B)
        wait_idx(A)
        start_gather(A)

        def pairstep(i, carry):
            j = 2 * i

            wait_gather(A)

            @pl.when(j + 1 < NCHUNK)
            def _():
                wait_idx(B)
                start_gather(B)
            scatter(A)

            @pl.when(j + 2 < NCHUNK)
            def _():
                start_idx(j + 2, A)

            @pl.when(j + 1 < NCHUNK)
            def _():
                wait_gather(B)

                @pl.when(j + 2 < NCHUNK)
                def _():
                    wait_idx(A)
                    start_gather(A)
                scatter(B)

                @pl.when(j + 3 < NCHUNK)
                def _():
                    start_idx(j + 3, B)
            return carry
        lax.fori_loop(0, (NCHUNK + 1) // 2, pairstep, 0)

        plsc.subcore_barrier()
        pltpu.sync_copy(acc_sh.at[pl.ds(base, RPT)],
                        acc_out.at[c].at[pl.ds(base, RPT)])

    return pl.kernel(
        body,
        out_type=[jax.ShapeDtypeStruct((NC, N, d), F32)],
        mesh=_MESH,
        scratch_types=[
            pltpu.VMEM((1, C), jnp.int32),
            pltpu.VMEM((1, C), jnp.int32),
            pltpu.VMEM((C, d), F32),
            pltpu.VMEM((1, C), jnp.int32),
            pltpu.VMEM((1, C), jnp.int32),
            pltpu.VMEM((C, d), F32),
            pltpu.VMEM_SHARED((N, d), F32),
            pltpu.SemaphoreType.DMA,
            pltpu.SemaphoreType.DMA,
            pltpu.SemaphoreType.DMA,
            pltpu.SemaphoreType.DMA,
        ],
    )


def _make_sc_count():
    def body(dst_hbm, zacc_hbm, cnt_out, dst0, dst1, ones_v, cnt_sh,
             si0, si1):
        c = lax.axis_index("c")
        s = lax.axis_index("s")
        wid = c * NS + s
        base = pl.multiple_of(jnp.minimum(s * 624, N - RPT), 8)

        pltpu.sync_copy(zacc_hbm.at[pl.ds(base, RPT)],
                        cnt_sh.at[pl.ds(base, RPT)])
        one16 = jnp.ones((16,), F32)

        def orow(r, carry):
            for l in range(8):
                ones_v[r, pl.ds(l * 16, 16)] = one16
            return carry
        lax.fori_loop(0, C, orow, 0)
        plsc.subcore_barrier()

        def start_idx(j, d_, si):
            pltpu.async_copy(dst_hbm.at[wid].at[j], d_, si)

        def wait_idx(d_, si):
            pltpu.make_async_copy(dst_hbm.at[wid].at[0], d_, si).wait()

        start_idx(0, dst0, si0)
        start_idx(1, dst1, si1)

        def pairstep(i, carry):
            j = 2 * i
            wait_idx(dst0, si0)
            pltpu.sync_copy(ones_v, cnt_sh.at[dst0.at[0]], add=True)

            @pl.when(j + 2 < NCHUNK)
            def _():
                start_idx(j + 2, dst0, si0)

            @pl.when(j + 1 < NCHUNK)
            def _():
                wait_idx(dst1, si1)
                pltpu.sync_copy(ones_v, cnt_sh.at[dst1.at[0]], add=True)

                @pl.when(j + 3 < NCHUNK)
                def _():
                    start_idx(j + 3, dst1, si1)
            return carry
        lax.fori_loop(0, (NCHUNK + 1) // 2, pairstep, 0)

        plsc.subcore_barrier()
        pltpu.sync_copy(cnt_sh.at[pl.ds(base, RPT)],
                        cnt_out.at[c].at[pl.ds(base, RPT)])

    return pl.kernel(
        body,
        out_type=[jax.ShapeDtypeStruct((NC, N, HIDDEN), F32)],
        mesh=_MESH,
        scratch_types=[
            pltpu.VMEM((1, C), jnp.int32),
            pltpu.VMEM((1, C), jnp.int32),
            pltpu.VMEM((C, HIDDEN), F32),
            pltpu.VMEM_SHARED((N, HIDDEN), F32),
            pltpu.SemaphoreType.DMA,
            pltpu.SemaphoreType.DMA,
        ],
    )


_sc_seg_h = _make_sc_segsum(HIDDEN)
_sc_seg_o = _make_sc_segsum(2 * N_CLASSES)
_sc_count = _make_sc_count()


@jax.jit
def kernel(x, edge_index, W1_l, b1_l, W1_r, W2_l, b2_l, W2_r):
    src3 = edge_index[0].reshape(NW, NCHUNK, 1, C)
    dst3 = edge_index[1].reshape(NW, NCHUNK, 1, C)

    y1, r1 = _tc_mm2(x, W1_l, W1_r, HIDDEN)
    zacc1 = jnp.zeros((N, HIDDEN), F32)
    cntp, = _sc_count(dst3, zacc1)
    acc1p, = _sc_seg_h(y1, src3, dst3, zacc1)

    yr2 = _tc_layer1_combine(acc1p, cntp, r1, b1_l, W2_l, W2_r)
    zacc2 = jnp.zeros((N, 2 * N_CLASSES), F32)
    acc2p, = _sc_seg_o(yr2, src3, dst3, zacc2)

    return _tc_final(acc2p, cntp, yr2, b2_l)


# speedup vs baseline: 10.6056x; 1.3088x over previous
import jax
import jax.numpy as jnp
from jax import lax
from jax.experimental import pallas as pl
from jax.experimental.pallas import tpu as pltpu
from jax.experimental.pallas import tpu_sc as plsc

N = 10000
E = 320000
D_IN = 128
HIDDEN = 128
N_CLASSES = 64

NC = 2
NS = 16
NW = NC * NS
EPW = E // NW
C = 80
NCHUNK = EPW // C
RPT = 640

F32 = jnp.float32


def _tc_mm2(x, wl, wr, d_out):
    def body(x_ref, wl_ref, wr_ref, y_ref, r_ref):
        xb = x_ref[...]
        y_ref[...] = jnp.dot(xb, wl_ref[...], preferred_element_type=F32)
        r_ref[...] = jnp.dot(xb, wr_ref[...], preferred_element_type=F32)

    return pl.pallas_call(
        body,
        out_shape=[jax.ShapeDtypeStruct((N, d_out), F32)] * 2,
    )(x, wl, wr)


def _tc_layer1_combine(accp, cntp, r1, b1, w2l, w2r):
    def body(a_ref, c_ref, r_ref, b_ref, wl_ref, wr_ref, yr_ref):
        a = a_ref[...]
        cc = c_ref[...]
        inv = 1.0 / jnp.maximum(cc[0, :, 0:1] + cc[1, :, 0:1], 1.0)
        h = (a[0] + a[1]) * inv + b_ref[...][None, :] + r_ref[...]
        h = jnp.maximum(h, 0.0)
        yr_ref[:, 0:N_CLASSES] = jnp.dot(h, wl_ref[...],
                                         preferred_element_type=F32)
        yr_ref[:, N_CLASSES:2 * N_CLASSES] = jnp.dot(
            h, wr_ref[...], preferred_element_type=F32)

    return pl.pallas_call(
        body,
        out_shape=jax.ShapeDtypeStruct((N, 2 * N_CLASSES), F32),
    )(accp, cntp, r1, b1, w2l, w2r)


def _tc_final(acc2p, cntp, yr2, b2):
    def body(a_ref, c_ref, yr_ref, b_ref, o_ref):
        a = a_ref[...]
        cc = c_ref[...]
        inv = 1.0 / jnp.maximum(cc[0, :, 0:1] + cc[1, :, 0:1], 1.0)
        agg = a[0, :, 0:N_CLASSES] + a[1, :, 0:N_CLASSES]
        r2 = yr_ref[:, N_CLASSES:2 * N_CLASSES]
        o = agg * inv + b_ref[...][None, :] + r2
        m = jnp.max(o, axis=1, keepdims=True)
        s = jnp.sum(jnp.exp(o - m), axis=1, keepdims=True)
        o_ref[...] = o - m - jnp.log(s)

    return pl.pallas_call(
        body,
        out_shape=jax.ShapeDtypeStruct((N, N_CLASSES), F32),
    )(acc2p, cntp, yr2, b2)


_MESH = plsc.VectorSubcoreMesh(
    core_axis_name="c", subcore_axis_name="s",
    num_cores=NC, num_subcores=NS)


def _make_sc_segsum(d):
    NR, NI = 3, 4

    def body(y_hbm, src_hbm, dst_hbm, zacc_hbm, acc_out, *refs):
        srcs = refs[0:NI]
        dsts = refs[NI:2 * NI]
        rows = refs[2 * NI:2 * NI + NR]
        dsc = refs[2 * NI + NR:2 * NI + 2 * NR]
        acc_sh = refs[2 * NI + 2 * NR]
        si = refs[2 * NI + 2 * NR + 1:2 * NI + 2 * NR + 1 + NI]
        sg = refs[2 * NI + 2 * NR + 1 + NI:2 * NI + 2 * NR + 1 + NI + NR]
        ss = refs[2 * NI + 2 * NR + 1 + NI + NR:]

        c = lax.axis_index("c")
        s = lax.axis_index("s")
        wid = c * NS + s
        base = pl.multiple_of(jnp.minimum(s * 624, N - RPT), 8)

        pltpu.sync_copy(zacc_hbm.at[pl.ds(base, RPT)],
                        acc_sh.at[pl.ds(base, RPT)])
        plsc.subcore_barrier()

        def start_idx(j, i):
            pltpu.async_copy(src_hbm.at[wid].at[j], srcs[i], si[i])
            pltpu.async_copy(dst_hbm.at[wid].at[j], dsts[i], si[i])

        def wait_idx(i):
            pltpu.make_async_copy(src_hbm.at[wid].at[0], srcs[i],
                                  si[i]).wait()
            pltpu.make_async_copy(dst_hbm.at[wid].at[0], dsts[i],
                                  si[i]).wait()

        def start_gather(r, i):
            pltpu.async_copy(y_hbm.at[srcs[i].at[0]], rows[r], sg[r])

        def wait_gather(r, i):
            pltpu.make_async_copy(y_hbm.at[srcs[i].at[0]], rows[r],
                                  sg[r]).wait()

        def wait_scatter(r):
            pltpu.make_async_copy(rows[r], acc_sh.at[dsc[r].at[0]],
                                  ss[r]).wait()

        for j0 in range(NI):
            start_idx(j0, j0)
        wait_idx(0)
        start_gather(0, 0)
        wait_idx(1)
        start_gather(1, 1)

        def phase(j, r, i):
            @pl.when(j < NCHUNK)
            def _():
                wait_gather(r, i)
                for kk in range(C // 16):
                    dsc[r][0, pl.ds(kk * 16, 16)] = (
                        dsts[i][0, pl.ds(kk * 16, 16)])
                pltpu.async_copy(rows[r], acc_sh.at[dsc[r].at[0]],
                                 ss[r], add=True)

                @pl.when(j + NI < NCHUNK)
                def _():
                    start_idx(j + NI, i)

                @pl.when(j + 2 < NCHUNK)
                def _():
                    r2 = (r + 2) % NR
                    i2 = (i + 2) % NI

                    @pl.when(j >= 1)
                    def _():
                        wait_scatter(r2)
                    wait_idx(i2)
                    start_gather(r2, i2)

        def bigstep(k, carry):
            jb = 12 * k
            for p in range(12):
                phase(jb + p, p % NR, p % NI)
            return carry
        lax.fori_loop(0, (NCHUNK + 11) // 12, bigstep, 0)

        for r in range(NR):
            wait_scatter(r)

        plsc.subcore_barrier()
        pltpu.sync_copy(acc_sh.at[pl.ds(base, RPT)],
                        acc_out.at[c].at[pl.ds(base, RPT)])

    return pl.kernel(
        body,
        out_type=[jax.ShapeDtypeStruct((NC, N, d), F32)],
        mesh=_MESH,
        scratch_types=(
            [pltpu.VMEM((1, C), jnp.int32)] * NI
            + [pltpu.VMEM((1, C), jnp.int32)] * NI
            + [pltpu.VMEM((C, d), F32)] * NR
            + [pltpu.VMEM((1, C), jnp.int32)] * NR
            + [pltpu.VMEM_SHARED((N, d), F32)]
            + [pltpu.SemaphoreType.DMA] * (NI + NR + NR)
        ),
    )


def _make_sc_count():
    def body(dst_hbm, zacc_hbm, cnt_out, dst0, dst1, ones_v, cnt_sh,
             si0, si1):
        c = lax.axis_index("c")
        s = lax.axis_index("s")
        wid = c * NS + s
        base = pl.multiple_of(jnp.minimum(s * 624, N - RPT), 8)

        pltpu.sync_copy(zacc_hbm.at[pl.ds(base, RPT)],
                        cnt_sh.at[pl.ds(base, RPT)])
        one16 = jnp.ones((16,), F32)

        def orow(r, carry):
            for l in range(8):
                ones_v[r, pl.ds(l * 16, 16)] = one16
            return carry
        lax.fori_loop(0, C, orow, 0)
        plsc.subcore_barrier()

        def start_idx(j, d_, si):
            pltpu.async_copy(dst_hbm.at[wid].at[j], d_, si)

        def wait_idx(d_, si):
            pltpu.make_async_copy(dst_hbm.at[wid].at[0], d_, si).wait()

        start_idx(0, dst0, si0)
        start_idx(1, dst1, si1)

        def pairstep(i, carry):
            j = 2 * i
            wait_idx(dst0, si0)
            pltpu.sync_copy(ones_v, cnt_sh.at[dst0.at[0]], add=True)

            @pl.when(j + 2 < NCHUNK)
            def _():
                start_idx(j + 2, dst0, si0)

            @pl.when(j + 1 < NCHUNK)
            def _():
                wait_idx(dst1, si1)
                pltpu.sync_copy(ones_v, cnt_sh.at[dst1.at[0]], add=True)

                @pl.when(j + 3 < NCHUNK)
                def _():
                    start_idx(j + 3, dst1, si1)
            return carry
        lax.fori_loop(0, (NCHUNK + 1) // 2, pairstep, 0)

        plsc.subcore_barrier()
        pltpu.sync_copy(cnt_sh.at[pl.ds(base, RPT)],
                        cnt_out.at[c].at[pl.ds(base, RPT)])

    return pl.kernel(
        body,
        out_type=[jax.ShapeDtypeStruct((NC, N, HIDDEN), F32)],
        mesh=_MESH,
        scratch_types=[
            pltpu.VMEM((1, C), jnp.int32),
            pltpu.VMEM((1, C), jnp.int32),
            pltpu.VMEM((C, HIDDEN), F32),
            pltpu.VMEM_SHARED((N, HIDDEN), F32),
            pltpu.SemaphoreType.DMA,
            pltpu.SemaphoreType.DMA,
        ],
    )


_sc_seg_h = _make_sc_segsum(HIDDEN)
_sc_seg_o = _make_sc_segsum(2 * N_CLASSES)
_sc_count = _make_sc_count()


@jax.jit
def kernel(x, edge_index, W1_l, b1_l, W1_r, W2_l, b2_l, W2_r):
    src3 = edge_index[0].reshape(NW, NCHUNK, 1, C)
    dst3 = edge_index[1].reshape(NW, NCHUNK, 1, C)

    y1, r1 = _tc_mm2(x, W1_l, W1_r, HIDDEN)
    zacc1 = jnp.zeros((N, HIDDEN), F32)
    cntp, = _sc_count(dst3, zacc1)
    acc1p, = _sc_seg_h(y1, src3, dst3, zacc1)

    yr2 = _tc_layer1_combine(acc1p, cntp, r1, b1_l, W2_l, W2_r)
    zacc2 = jnp.zeros((N, 2 * N_CLASSES), F32)
    acc2p, = _sc_seg_o(yr2, src3, dst3, zacc2)

    return _tc_final(acc2p, cntp, yr2, b2_l)


# speedup vs baseline: 10.7008x; 1.0090x over previous
import jax
import jax.numpy as jnp
from jax import lax
from jax.experimental import pallas as pl
from jax.experimental.pallas import tpu as pltpu
from jax.experimental.pallas import tpu_sc as plsc

N = 10000
E = 320000
D_IN = 128
HIDDEN = 128
N_CLASSES = 64

NC = 2
NS = 16
NW = NC * NS
EPW = E // NW
C = 80
NCHUNK = EPW // C
RPT = 640

F32 = jnp.float32


def _tc_mm2(x, wl, wr, d_out):
    def body(x_ref, wl_ref, wr_ref, y_ref, r_ref):
        xb = x_ref[...]
        y_ref[...] = jnp.dot(xb, wl_ref[...], preferred_element_type=F32)
        r_ref[...] = jnp.dot(xb, wr_ref[...], preferred_element_type=F32)

    return pl.pallas_call(
        body,
        out_shape=[jax.ShapeDtypeStruct((N, d_out), F32)] * 2,
    )(x, wl, wr)


def _tc_layer1_combine(accp, cntp, r1, b1, w2l, w2r):
    def body(a_ref, c_ref, r_ref, b_ref, wl_ref, wr_ref, yr_ref):
        a = a_ref[...]
        cc = c_ref[...]
        inv = 1.0 / jnp.maximum(cc[0, :, 0:1] + cc[1, :, 0:1], 1.0)
        h = (a[0] + a[1]) * inv + b_ref[...][None, :] + r_ref[...]
        h = jnp.maximum(h, 0.0)
        yr_ref[:, 0:N_CLASSES] = jnp.dot(h, wl_ref[...],
                                         preferred_element_type=F32)
        yr_ref[:, N_CLASSES:2 * N_CLASSES] = jnp.dot(
            h, wr_ref[...], preferred_element_type=F32)

    return pl.pallas_call(
        body,
        out_shape=jax.ShapeDtypeStruct((N, 2 * N_CLASSES), F32),
    )(accp, cntp, r1, b1, w2l, w2r)


def _tc_final(acc2p, cntp, yr2, b2):
    def body(a_ref, c_ref, yr_ref, b_ref, o_ref):
        a = a_ref[...]
        cc = c_ref[...]
        inv = 1.0 / jnp.maximum(cc[0, :, 0:1] + cc[1, :, 0:1], 1.0)
        agg = a[0, :, 0:N_CLASSES] + a[1, :, 0:N_CLASSES]
        r2 = yr_ref[:, N_CLASSES:2 * N_CLASSES]
        o = agg * inv + b_ref[...][None, :] + r2
        m = jnp.max(o, axis=1, keepdims=True)
        s = jnp.sum(jnp.exp(o - m), axis=1, keepdims=True)
        o_ref[...] = o - m - jnp.log(s)

    return pl.pallas_call(
        body,
        out_shape=jax.ShapeDtypeStruct((N, N_CLASSES), F32),
    )(acc2p, cntp, yr2, b2)


_MESH = plsc.VectorSubcoreMesh(
    core_axis_name="c", subcore_axis_name="s",
    num_cores=NC, num_subcores=NS)


def _make_sc_segsum(d):
    NR, NI = 3, 4

    def body(y_hbm, src_hbm, dst_hbm, zacc_hbm, acc_out, *refs):
        srcs = refs[0:NI]
        dsts = refs[NI:2 * NI]
        rows = refs[2 * NI:2 * NI + NR]
        dsc = refs[2 * NI + NR:2 * NI + 2 * NR]
        acc_sh = refs[2 * NI + 2 * NR]
        si = refs[2 * NI + 2 * NR + 1:2 * NI + 2 * NR + 1 + NI]
        sg = refs[2 * NI + 2 * NR + 1 + NI:2 * NI + 2 * NR + 1 + NI + NR]
        ss = refs[2 * NI + 2 * NR + 1 + NI + NR:]

        c = lax.axis_index("c")
        s = lax.axis_index("s")
        wid = c * NS + s
        base = pl.multiple_of(jnp.minimum(s * 624, N - RPT), 8)

        pltpu.sync_copy(zacc_hbm.at[pl.ds(base, RPT)],
                        acc_sh.at[pl.ds(base, RPT)])
        plsc.subcore_barrier()

        def start_idx(j, i):
            pltpu.async_copy(src_hbm.at[wid].at[j], srcs[i], si[i])
            pltpu.async_copy(dst_hbm.at[wid].at[j], dsts[i], si[i])

        def wait_idx(i):
            pltpu.make_async_copy(src_hbm.at[wid].at[0], srcs[i],
                                  si[i]).wait()
            pltpu.make_async_copy(dst_hbm.at[wid].at[0], dsts[i],
                                  si[i]).wait()

        def start_gather(r, i):
            pltpu.async_copy(y_hbm.at[srcs[i].at[0]], rows[r], sg[r])

        def wait_gather(r, i):
            pltpu.make_async_copy(y_hbm.at[srcs[i].at[0]], rows[r],
                                  sg[r]).wait()

        def wait_scatter(r):
            pltpu.make_async_copy(rows[r], acc_sh.at[dsc[r].at[0]],
                                  ss[r]).wait()

        for j0 in range(NI):
            start_idx(j0, j0)
        wait_idx(0)
        start_gather(0, 0)
        wait_idx(1)
        start_gather(1, 1)

        def phase(j, r, i):
            @pl.when(j < NCHUNK)
            def _():
                wait_gather(r, i)
                for kk in range(C // 16):
                    dsc[r][0, pl.ds(kk * 16, 16)] = (
                        dsts[i][0, pl.ds(kk * 16, 16)])
                pltpu.async_copy(rows[r], acc_sh.at[dsc[r].at[0]],
                                 ss[r], add=True)

                @pl.when(j + NI < NCHUNK)
                def _():
                    start_idx(j + NI, i)

                @pl.when(j + 2 < NCHUNK)
                def _():
                    r2 = (r + 2) % NR
                    i2 = (i + 2) % NI

                    @pl.when(j >= 1)
                    def _():
                        wait_scatter(r2)
                    wait_idx(i2)
                    start_gather(r2, i2)

        def bigstep(k, carry):
            jb = 12 * k
            for p in range(12):
                phase(jb + p, p % NR, p % NI)
            return carry
        lax.fori_loop(0, (NCHUNK + 11) // 12, bigstep, 0)

        for r in range(NR):
            wait_scatter(r)

        plsc.subcore_barrier()
        pltpu.sync_copy(acc_sh.at[pl.ds(base, RPT)],
                        acc_out.at[c].at[pl.ds(base, RPT)])

    return pl.kernel(
        body,
        out_type=[jax.ShapeDtypeStruct((NC, N, d), F32)],
        mesh=_MESH,
        scratch_types=(
            [pltpu.VMEM((1, C), jnp.int32)] * NI
            + [pltpu.VMEM((1, C), jnp.int32)] * NI
            + [pltpu.VMEM((C, d), F32)] * NR
            + [pltpu.VMEM((1, C), jnp.int32)] * NR
            + [pltpu.VMEM_SHARED((N, d), F32)]
            + [pltpu.SemaphoreType.DMA] * (NI + NR + NR)
        ),
    )


def _make_sc_count():
    NR, NI = 3, 4

    def body(dst_hbm, zacc_hbm, cnt_out, *refs):
        dsts = refs[0:NI]
        dsc = refs[NI:NI + NR]
        ones_v = refs[NI + NR]
        cnt_sh = refs[NI + NR + 1]
        si = refs[NI + NR + 2:NI + NR + 2 + NI]
        ss = refs[NI + NR + 2 + NI:]

        c = lax.axis_index("c")
        s = lax.axis_index("s")
        wid = c * NS + s
        base = pl.multiple_of(jnp.minimum(s * 624, N - RPT), 8)

        pltpu.sync_copy(zacc_hbm.at[pl.ds(base, RPT)],
                        cnt_sh.at[pl.ds(base, RPT)])
        one16 = jnp.ones((16,), F32)

        def orow(r, carry):
            for l in range(8):
                ones_v[r, pl.ds(l * 16, 16)] = one16
            return carry
        lax.fori_loop(0, C, orow, 0)
        plsc.subcore_barrier()

        def start_idx(j, i):
            pltpu.async_copy(dst_hbm.at[wid].at[j], dsts[i], si[i])

        def wait_idx(i):
            pltpu.make_async_copy(dst_hbm.at[wid].at[0], dsts[i],
                                  si[i]).wait()

        def wait_scatter(r):
            pltpu.make_async_copy(ones_v, cnt_sh.at[dsc[r].at[0]],
                                  ss[r]).wait()

        for j0 in range(NI):
            start_idx(j0, j0)

        def phase(j, r, i):
            @pl.when(j < NCHUNK)
            def _():
                @pl.when(j >= NR)
                def _():
                    wait_scatter(r)
                wait_idx(i)
                for kk in range(C // 16):
                    dsc[r][0, pl.ds(kk * 16, 16)] = (
                        dsts[i][0, pl.ds(kk * 16, 16)])
                pltpu.async_copy(ones_v, cnt_sh.at[dsc[r].at[0]],
                                 ss[r], add=True)

                @pl.when(j + NI < NCHUNK)
                def _():
                    start_idx(j + NI, i)

        def bigstep(k, carry):
            jb = 12 * k
            for p in range(12):
                phase(jb + p, p % NR, p % NI)
            return carry
        lax.fori_loop(0, (NCHUNK + 11) // 12, bigstep, 0)
        for r in range(NR):
            wait_scatter(r)

        plsc.subcore_barrier()
        pltpu.sync_copy(cnt_sh.at[pl.ds(base, RPT)],
                        cnt_out.at[c].at[pl.ds(base, RPT)])

    return pl.kernel(
        body,
        out_type=[jax.ShapeDtypeStruct((NC, N, HIDDEN), F32)],
        mesh=_MESH,
        scratch_types=(
            [pltpu.VMEM((1, C), jnp.int32)] * NI
            + [pltpu.VMEM((1, C), jnp.int32)] * NR
            + [pltpu.VMEM((C, HIDDEN), F32)]
            + [pltpu.VMEM_SHARED((N, HIDDEN), F32)]
            + [pltpu.SemaphoreType.DMA] * (NI + NR)
        ),
    )


_sc_seg_h = _make_sc_segsum(HIDDEN)
_sc_seg_o = _make_sc_segsum(2 * N_CLASSES)
_sc_count = _make_sc_count()


@jax.jit
def kernel(x, edge_index, W1_l, b1_l, W1_r, W2_l, b2_l, W2_r):
    src3 = edge_index[0].reshape(NW, NCHUNK, 1, C)
    dst3 = edge_index[1].reshape(NW, NCHUNK, 1, C)

    y1, r1 = _tc_mm2(x, W1_l, W1_r, HIDDEN)
    zacc1 = jnp.zeros((N, HIDDEN), F32)
    cntp, = _sc_count(dst3, zacc1)
    acc1p, = _sc_seg_h(y1, src3, dst3, zacc1)

    yr2 = _tc_layer1_combine(acc1p, cntp, r1, b1_l, W2_l, W2_r)
    zacc2 = jnp.zeros((N, 2 * N_CLASSES), F32)
    acc2p, = _sc_seg_o(yr2, src3, dst3, zacc2)

    return _tc_final(acc2p, cntp, yr2, b2_l)
